# Initial kernel scaffold; baseline (speedup 1.0000x reference)
import jax
import jax.numpy as jnp
from jax.experimental import pallas as pl


def kernel(X, indices, W_self, b_self, W_neigh, b_neigh):
    raise NotImplementedError("write your pallas kernel here")


# speedup vs baseline: 5.3180x; 5.3180x over previous
import functools

import jax
import jax.numpy as jnp
from jax import lax
from jax.experimental import pallas as pl
from jax.experimental.pallas import tpu as pltpu
from jax.experimental.pallas import tpu_sc as plsc

N = 10000
E = 320000
D = 128
NC = 2
NS = 16
NW = NC * NS
C = 128
CH = 79
EPW = C * CH
EP = EPW * NW
RPT = 640
NPAD = RPT * NS

_mesh = plsc.VectorSubcoreMesh(core_axis_name="c", subcore_axis_name="s")


@functools.partial(
    pl.kernel,
    mesh=_mesh,
    out_type=[
        jax.ShapeDtypeStruct((NW, RPT, D), jnp.float32),
        jax.ShapeDtypeStruct((NW, RPT), jnp.float32),
    ],
    scratch_types=[
        pltpu.VMEM((CH, C), jnp.int32),
        pltpu.VMEM((CH, C), jnp.int32),
        pltpu.VMEM((C, D), jnp.float32),
        pltpu.VMEM((C,), jnp.float32),
        pltpu.VMEM((RPT,), jnp.float32),
        pltpu.VMEM_SHARED((NPAD, D), jnp.float32),
        pltpu.VMEM_SHARED((NPAD,), jnp.float32),
        pltpu.SemaphoreType.DMA,
    ],
)
def _sc_scatter(x_hbm, src_hbm, dst_hbm, sums_hbm, deg_hbm,
                srcb, dstb, rowb, onesb, zb, shared, shared_deg, sem):
    c = lax.axis_index("c")
    s = lax.axis_index("s")
    wid = s * NC + c
    oid = c * NS + s
    base = s * RPT

    z16 = jnp.zeros((16,), jnp.float32)
    ones16 = jnp.ones((16,), jnp.float32)

    for g in range(C // 16):
        onesb[pl.ds(g * 16, 16)] = ones16

    def zero_zb(i, carry):
        zb[pl.ds(i * 16, 16)] = z16
        return carry
    lax.fori_loop(0, RPT // 16, zero_zb, 0)

    def zero_row(r, carry):
        for g in range(D // 16):
            rowb[r, pl.ds(g * 16, 16)] = z16
        return carry
    lax.fori_loop(0, C, zero_row, 0)

    for t in range(RPT // C):
        pltpu.sync_copy(rowb, shared.at[pl.ds(base + t * C, C)])
    pltpu.sync_copy(zb, shared_deg.at[pl.ds(base, RPT)])

    pltpu.sync_copy(src_hbm.at[wid], srcb)
    pltpu.sync_copy(dst_hbm.at[wid], dstb)

    plsc.subcore_barrier()

    def edge_chunk(j, carry):
        pltpu.async_copy(x_hbm.at[dstb.at[j]], rowb, sem).wait()
        pltpu.sync_copy(rowb, shared.at[srcb.at[j]], add=True)
        pltpu.sync_copy(onesb, shared_deg.at[srcb.at[j]], add=True)
        return carry
    lax.fori_loop(0, CH, edge_chunk, 0)

    plsc.subcore_barrier()

    pltpu.sync_copy(shared.at[pl.ds(base, RPT)], sums_hbm.at[oid])
    pltpu.sync_copy(shared_deg.at[pl.ds(base, RPT)], deg_hbm.at[oid])


def _tc_body(x_ref, p_ref, dg_ref, ws_ref, bs_ref, wn_ref, bn_ref, o_ref):
    x = x_ref[...]
    hs = lax.dot_general(x, ws_ref[...], (((1,), (1,)), ((), ())),
                         preferred_element_type=jnp.float32) + bs_ref[...]
    p = p_ref[0] + p_ref[1]
    dg = dg_ref[...]
    d = dg[0] + dg[1] + 1e-9
    pn = p / d
    hn = lax.dot_general(pn, wn_ref[...], (((1,), (1,)), ((), ())),
                         preferred_element_type=jnp.float32) + bn_ref[...]
    o_ref[...] = jnp.tanh(jnp.concatenate([hs, hn], axis=1))


def kernel(X, indices, W_self, b_self, W_neigh, b_neigh):
    src = indices[0].astype(jnp.int32)
    dst = indices[1].astype(jnp.int32)
    pad = EP - E
    srcp = jnp.concatenate([src, jnp.full((pad,), N, jnp.int32)]).reshape(NW, CH, C)
    dstp = jnp.concatenate([dst, jnp.zeros((pad,), jnp.int32)]).reshape(NW, CH, C)

    sums, degs = _sc_scatter(X, srcp, dstp)
    psums = sums.reshape(NC, NPAD, D)
    pdegs = degs.reshape(NC, NPAD)[:, :, None]

    BR = 1000
    out = pl.pallas_call(
        _tc_body,
        grid=(N // BR,),
        in_specs=[
            pl.BlockSpec((BR, D), lambda i: (i, 0)),
            pl.BlockSpec((NC, BR, D), lambda i: (0, i, 0)),
            pl.BlockSpec((NC, BR, 1), lambda i: (0, i, 0)),
            pl.BlockSpec((D, D), lambda i: (0, 0)),
            pl.BlockSpec((1, D), lambda i: (0, 0)),
            pl.BlockSpec((D, D), lambda i: (0, 0)),
            pl.BlockSpec((1, D), lambda i: (0, 0)),
        ],
        out_specs=pl.BlockSpec((BR, 2 * D), lambda i: (i, 0)),
        out_shape=jax.ShapeDtypeStruct((N, 2 * D), jnp.float32),
    )(X, psums, pdegs, W_self, b_self.reshape(1, D), W_neigh,
      b_neigh.reshape(1, D))
    return out
